# baseline (device time: 25652 ns/iter reference)
import jax
import jax.numpy as jnp
from jax import lax
from jax.experimental import pallas as pl
from jax.experimental.pallas import tpu as pltpu

NCH = 8


def kernel(x, dy):
    k, d = x.shape
    _, f = dy.shape
    half = d // 2
    fx = f // 2
    cw = fx // NCH

    def body(x_ref, dy_ref, out_ref,
             dyblk, pfull_buf, yrecv_buf, sred_buf,
             dy_sem, out_sems, ysend_sems, yrecv_sems, xsend_sems, xrecv_sems):
        my_x = lax.axis_index("x")
        my_y = lax.axis_index("y")
        my_z = lax.axis_index("z")
        ypartner = (my_x, 1 - my_y, my_z)
        xpartner = (1 - my_x, my_y, my_z)

        mine = my_y * half
        theirs = (1 - my_y) * half
        col0 = my_x * fx

        dy_copy = pltpu.make_async_copy(
            dy_ref.at[:, pl.ds(col0, fx)], dyblk, dy_sem
        )
        dy_copy.start()

        barrier = pltpu.get_barrier_semaphore()
        for nbr in (ypartner, xpartner):
            pl.semaphore_signal(
                barrier, inc=1, device_id=nbr,
                device_id_type=pl.DeviceIdType.MESH,
            )
        pl.semaphore_wait(barrier, 2)
        dy_copy.wait()

        def y_rdma(j):
            return pltpu.make_async_remote_copy(
                src_ref=pfull_buf.at[j, pl.ds(theirs, half), :],
                dst_ref=yrecv_buf.at[j],
                send_sem=ysend_sems.at[j],
                recv_sem=yrecv_sems.at[j],
                device_id=ypartner,
                device_id_type=pl.DeviceIdType.MESH,
            )

        def x_rdma(j):
            return pltpu.make_async_remote_copy(
                src_ref=sred_buf.at[j],
                dst_ref=out_ref.at[:, pl.ds(col0 + j * cw, cw)],
                send_sem=xsend_sems.at[j],
                recv_sem=xrecv_sems.at[j],
                device_id=xpartner,
                device_id_type=pl.DeviceIdType.MESH,
            )

        for j in range(NCH):
            pfull_buf[j] = lax.dot_general(
                x_ref[...], dyblk[:, j * cw:(j + 1) * cw],
                dimension_numbers=(((0,), (0,)), ((), ())),
                preferred_element_type=jnp.float32,
            )
            y_rdma(j).start()

        for j in range(NCH):
            y_rdma(j).wait_recv()
            sred_buf[j] = pfull_buf[j, pl.ds(mine, half), :] + yrecv_buf[j]
            x_rdma(j).start()
            pltpu.make_async_copy(
                sred_buf.at[j],
                out_ref.at[:, pl.ds(col0 + j * cw, cw)],
                out_sems.at[j],
            ).start()

        for j in range(NCH):
            x_rdma(j).wait_recv()
            pltpu.make_async_copy(
                sred_buf.at[j],
                out_ref.at[:, pl.ds(col0 + j * cw, cw)],
                out_sems.at[j],
            ).wait()
            y_rdma(j).wait_send()
            x_rdma(j).wait_send()

    return pl.pallas_call(
        body,
        out_shape=jax.ShapeDtypeStruct((half, f), jnp.float32),
        in_specs=[
            pl.BlockSpec(memory_space=pltpu.VMEM),
            pl.BlockSpec(memory_space=pl.ANY),
        ],
        out_specs=pl.BlockSpec(memory_space=pltpu.VMEM),
        scratch_shapes=[
            pltpu.VMEM((k, fx), jnp.float32),
            pltpu.VMEM((NCH, d, cw), jnp.float32),
            pltpu.VMEM((NCH, half, cw), jnp.float32),
            pltpu.VMEM((NCH, half, cw), jnp.float32),
            pltpu.SemaphoreType.DMA,
            pltpu.SemaphoreType.DMA((NCH,)),
            pltpu.SemaphoreType.DMA((NCH,)),
            pltpu.SemaphoreType.DMA((NCH,)),
            pltpu.SemaphoreType.DMA((NCH,)),
            pltpu.SemaphoreType.DMA((NCH,)),
        ],
        compiler_params=pltpu.CompilerParams(collective_id=0),
    )(x, dy)


# device time: 20255 ns/iter; 1.2665x vs baseline; 1.2665x over previous
import jax
import jax.numpy as jnp
from jax import lax
from jax.experimental import pallas as pl
from jax.experimental.pallas import tpu as pltpu

NCH = 16


def kernel(x, dy):
    k, d = x.shape
    _, f = dy.shape
    half = d // 2
    cw = f // NCH

    def body(x_ref, dy_ref, out_ref,
             pfull_buf, psend_buf, yrecv_buf,
             ysend_sems, yrecv_sems):
        my_x = lax.axis_index("x")
        my_y = lax.axis_index("y")
        my_z = lax.axis_index("z")
        ypartner = (my_x, 1 - my_y, my_z)

        mine = my_y * half
        theirs = (1 - my_y) * half

        barrier = pltpu.get_barrier_semaphore()
        pl.semaphore_signal(
            barrier, inc=1, device_id=ypartner,
            device_id_type=pl.DeviceIdType.MESH,
        )
        pl.semaphore_wait(barrier, 1)

        def y_rdma(j):
            return pltpu.make_async_remote_copy(
                src_ref=psend_buf.at[j, pl.ds(theirs, half), :],
                dst_ref=yrecv_buf.at[j],
                send_sem=ysend_sems.at[j],
                recv_sem=yrecv_sems.at[j],
                device_id=ypartner,
                device_id_type=pl.DeviceIdType.MESH,
            )

        for j in range(NCH):
            p = lax.dot_general(
                x_ref[...], dy_ref[:, j * cw:(j + 1) * cw],
                dimension_numbers=(((0,), (0,)), ((), ())),
                preferred_element_type=jnp.float32,
            )
            pfull_buf[j] = p
            psend_buf[j] = p.astype(jnp.bfloat16)
            y_rdma(j).start()

        for j in range(NCH):
            y_rdma(j).wait_recv()
            out_ref[:, j * cw:(j + 1) * cw] = (
                pfull_buf[j, pl.ds(mine, half), :]
                + yrecv_buf[j].astype(jnp.float32)
            )

        for j in range(NCH):
            y_rdma(j).wait_send()

    return pl.pallas_call(
        body,
        out_shape=jax.ShapeDtypeStruct((half, f), jnp.float32),
        in_specs=[
            pl.BlockSpec(memory_space=pltpu.VMEM),
            pl.BlockSpec(memory_space=pltpu.VMEM),
        ],
        out_specs=pl.BlockSpec(memory_space=pltpu.VMEM),
        scratch_shapes=[
            pltpu.VMEM((NCH, d, cw), jnp.float32),
            pltpu.VMEM((NCH, d, cw), jnp.bfloat16),
            pltpu.VMEM((NCH, half, cw), jnp.bfloat16),
            pltpu.SemaphoreType.DMA((NCH,)),
            pltpu.SemaphoreType.DMA((NCH,)),
        ],
        compiler_params=pltpu.CompilerParams(collective_id=0),
    )(x, dy)
